# baseline (speedup 1.0000x reference)
import functools

import jax
import jax.numpy as jnp
from jax import lax
from jax.experimental import pallas as pl
from jax.experimental.pallas import tpu as pltpu
from jax.experimental.pallas import tpu_sc as plsc

_B, _T, _D = 4, 2048, 128
_NC, _NS, _L = 2, 16, 16
_NW = _NC * _NS
_CW = _B * _T * _D // _NW
_PERIOD = _T * _D


@jax.jit
def _sc_add(x_flat, pos_flat):
    mesh = plsc.VectorSubcoreMesh(core_axis_name="c", subcore_axis_name="s")

    @functools.partial(
        pl.kernel,
        out_type=jax.ShapeDtypeStruct((_B * _T * _D,), jnp.float32),
        mesh=mesh,
        scratch_types=[
            pltpu.VMEM((_CW,), jnp.float32),
            pltpu.VMEM((_CW,), jnp.float32),
            pltpu.SemaphoreType.DMA,
            pltpu.SemaphoreType.DMA,
            pltpu.SemaphoreType.DMA,
        ],
    )
    def k(x_hbm, pos_hbm, out_hbm, pos_v, buf_v, sp, sx, so):
        wid = lax.axis_index("s") * _NC + lax.axis_index("c")
        base = wid * _CW
        pbase = lax.rem(base, _PERIOD)
        xload = pltpu.async_copy(x_hbm.at[pl.ds(base, _CW)], buf_v, sx)
        pload = pltpu.async_copy(pos_hbm.at[pl.ds(pbase, _CW)], pos_v, sp)
        xload.wait()
        pload.wait()
        stores = []
        qs = _CW // 4
        for q in range(4):
            def body(i):
                plsc.addupdate(buf_v.at[pl.ds(i, _L)],
                               pos_v.at[pl.ds(i, _L)][...])

            plsc.parallel_loop(q * qs, (q + 1) * qs, _L, unroll=8)(body)
            stores.append(
                pltpu.async_copy(buf_v.at[pl.ds(q * qs, qs)],
                                 out_hbm.at[pl.ds(base + q * qs, qs)], so))
        for st in stores:
            st.wait()

    return k(x_flat, pos_flat)


def kernel(x, pos_table):
    out = _sc_add(x.reshape(-1), pos_table.reshape(-1))
    return out.reshape(_B, _T, _D)


# speedup vs baseline: 1.0148x; 1.0148x over previous
import functools

import jax
import jax.numpy as jnp
from jax import lax
from jax.experimental import pallas as pl
from jax.experimental.pallas import tpu as pltpu
from jax.experimental.pallas import tpu_sc as plsc

_B, _T, _D = 4, 2048, 128
_NC, _NS, _L = 2, 16, 16
_NW = _NC * _NS
_CHUNK = _T * _D // _NW
_H = _CHUNK // 2


@jax.jit
def _sc_add(x_flat, pos_flat):
    mesh = plsc.VectorSubcoreMesh(core_axis_name="c", subcore_axis_name="s")

    @functools.partial(
        pl.kernel,
        out_type=jax.ShapeDtypeStruct((_B * _T * _D,), jnp.float32),
        mesh=mesh,
        scratch_types=[
            pltpu.VMEM((_CHUNK,), jnp.float32),
            pltpu.VMEM((_B, _CHUNK), jnp.float32),
            pltpu.SemaphoreType.DMA,
            pltpu.SemaphoreType.DMA,
            pltpu.SemaphoreType.DMA,
            pltpu.SemaphoreType.DMA,
            pltpu.SemaphoreType.DMA,
            pltpu.SemaphoreType.DMA,
        ],
    )
    def k(x_hbm, pos_hbm, out_hbm, pos_v, bufs, sp, s0, s1, s2, s3, so):
        isems = (s0, s1, s2, s3)
        wid = lax.axis_index("s") * _NC + lax.axis_index("c")
        pbase = wid * _CHUNK
        pload = pltpu.async_copy(pos_hbm.at[pl.ds(pbase, _CHUNK)], pos_v, sp)
        loads = []
        for b in range(_B):
            base = b * _T * _D + pbase
            loads.append(
                pltpu.async_copy(x_hbm.at[pl.ds(base, _CHUNK)],
                                 bufs.at[b], isems[b]))
        pload.wait()
        stores = []
        for b in range(_B):
            loads[b].wait()
            xb = bufs.at[b]
            base = b * _T * _D + pbase
            for h in range(2):
                def body(i, xb=xb):
                    plsc.addupdate(xb.at[pl.ds(i, _L)],
                                   pos_v.at[pl.ds(i, _L)][...])

                plsc.parallel_loop(h * _H, (h + 1) * _H, _L, unroll=16)(body)
                stores.append(
                    pltpu.async_copy(xb.at[pl.ds(h * _H, _H)],
                                     out_hbm.at[pl.ds(base + h * _H, _H)],
                                     so))
        for st in stores:
            st.wait()

    return k(x_flat, pos_flat)


def kernel(x, pos_table):
    out = _sc_add(x.reshape(-1), pos_table.reshape(-1))
    return out.reshape(_B, _T, _D)


# speedup vs baseline: 1.0482x; 1.0329x over previous
import functools

import jax
import jax.numpy as jnp
from jax import lax
from jax.experimental import pallas as pl
from jax.experimental.pallas import tpu as pltpu
from jax.experimental.pallas import tpu_sc as plsc

_B, _T, _D = 4, 2048, 128
_NC, _NS, _L = 2, 16, 16
_NW = _NC * _NS
_R = _T // _NW


@jax.jit
def _sc_add(x, pos_table):
    mesh = plsc.VectorSubcoreMesh(core_axis_name="c", subcore_axis_name="s")

    @functools.partial(
        pl.kernel,
        out_type=jax.ShapeDtypeStruct((_B, _T, _D), jnp.float32),
        mesh=mesh,
        scratch_types=[
            pltpu.VMEM((_R, _D), jnp.float32),
            pltpu.VMEM((_B, _R, _D), jnp.float32),
            pltpu.SemaphoreType.DMA,
            pltpu.SemaphoreType.DMA,
            pltpu.SemaphoreType.DMA,
            pltpu.SemaphoreType.DMA,
            pltpu.SemaphoreType.DMA,
            pltpu.SemaphoreType.DMA,
        ],
    )
    def k(x_hbm, pos_hbm, out_hbm, pos_v, bufs, sp, s0, s1, s2, s3, so):
        isems = (s0, s1, s2, s3)
        wid = lax.axis_index("s") * _NC + lax.axis_index("c")
        row0 = wid * _R
        pload = pltpu.async_copy(pos_hbm.at[pl.ds(row0, _R), :], pos_v, sp)
        loads = []
        for b in range(_B):
            loads.append(
                pltpu.async_copy(x_hbm.at[b, pl.ds(row0, _R), :],
                                 bufs.at[b], isems[b]))
        pload.wait()
        stores = []
        for b in range(_B):
            loads[b].wait()
            xb = bufs.at[b]

            def body(r, xb=xb):
                for c in range(0, _D, _L):
                    plsc.addupdate(xb.at[pl.ds(r, 1), pl.ds(c, _L)],
                                   pos_v.at[pl.ds(r, 1), pl.ds(c, _L)][...])

            plsc.parallel_loop(0, _R, 1, unroll=2)(body)
            stores.append(
                pltpu.async_copy(xb, out_hbm.at[b, pl.ds(row0, _R), :], so))
        for st in stores:
            st.wait()

    return k(x, pos_table)


def kernel(x, pos_table):
    return _sc_add(x, pos_table)
